# baseline (speedup 1.0000x reference)
import functools

import jax
import jax.numpy as jnp
from jax.experimental import pallas as pl
from jax.experimental.pallas import tpu as pltpu
EPS = 1e-5
LANE = 128


def _conv_stats_kernel(x_ref, w_ref, m_ref, y_ref, stats_ref,
                       *, KH, KW, W, n_rows, pad_rows, CIN, PB):
    m = m_ref[...]
    for i in range(PB):
        xb = x_ref[i].astype(jnp.bfloat16)
        if pad_rows:
            xb = jnp.concatenate(
                [xb, jnp.zeros((pad_rows, CIN), jnp.bfloat16)], axis=0)
        taps = []
        for kh in range(KH):
            for kw in range(KW):
                off = kh * W + kw
                taps.append(xb[off:off + n_rows, :])
        patches = jnp.concatenate(taps, axis=-1)
        y = jnp.dot(patches, w_ref[...],
                    preferred_element_type=jnp.float32)
        y_ref[i] = y.astype(jnp.bfloat16).reshape(n_rows // W, W, -1)
        stats_ref[i, 0:1, :] = jnp.dot(m, y,
                                       preferred_element_type=jnp.float32)
        stats_ref[i, 1:2, :] = jnp.dot(m, y * y,
                                       preferred_element_type=jnp.float32)


def _bn_relu_kernel(y_ref, stats_ref, g_ref, b_ref, o_ref,
                    *, eps, inv_count, OW):
    tot = jnp.sum(stats_ref[...], axis=0)
    mean = tot[0:1, :] * inv_count
    var = tot[1:2, :] * inv_count - mean * mean
    inv_std = jax.lax.rsqrt(var + eps)
    scale = (g_ref[...] * inv_std).reshape(1, 1, 1, -1)
    shift = (b_ref[...] - mean * g_ref[...] * inv_std).reshape(1, 1, 1, -1)
    z = y_ref[...].astype(jnp.float32)
    z = jnp.maximum(z * scale + shift, 0.0)
    o_ref[...] = jnp.transpose(z, (1, 2, 0, 3))[:, :OW]


@jax.jit
def _conv_bn_relu(x_nchw, w_oihw, gamma, beta):
    N, CIN, H, W = x_nchw.shape
    COUT, _, KH, KW = w_oihw.shape
    OH, OW = H - KH + 1, W - KW + 1
    C_PAD = ((COUT + LANE - 1) // LANE) * LANE
    n_rows = OH * W
    HWP = -(-(H * W + KW - 1) // 8) * 8

    x = jnp.transpose(x_nchw, (0, 2, 3, 1)).reshape(N, H * W, CIN)
    w = jnp.transpose(w_oihw, (2, 3, 1, 0)).reshape(KH * KW * CIN, COUT)
    w = jnp.pad(w.astype(jnp.bfloat16), ((0, 0), (0, C_PAD - COUT)))
    g = jnp.pad(gamma.astype(jnp.float32), (0, C_PAD - COUT)).reshape(1, C_PAD)
    b = jnp.pad(beta.astype(jnp.float32), (0, C_PAD - COUT)).reshape(1, C_PAD)
    mask = (jnp.arange(n_rows) % W < OW).astype(jnp.float32).reshape(1, n_rows)

    PB = 2 if N % 2 == 0 else 1
    y, stats = pl.pallas_call(
        functools.partial(_conv_stats_kernel, KH=KH, KW=KW, W=W,
                          n_rows=n_rows, pad_rows=HWP - H * W, CIN=CIN, PB=PB),
        grid=(N // PB,),
        in_specs=[
            pl.BlockSpec((PB, H * W, CIN), lambda n: (n, 0, 0)),
            pl.BlockSpec((KH * KW * CIN, C_PAD), lambda n: (0, 0)),
            pl.BlockSpec((1, n_rows), lambda n: (0, 0)),
        ],
        out_specs=(
            pl.BlockSpec((PB, OH, W, C_PAD), lambda n: (n, 0, 0, 0)),
            pl.BlockSpec((PB, 2, C_PAD), lambda n: (n, 0, 0)),
        ),
        out_shape=(
            jax.ShapeDtypeStruct((N, OH, W, C_PAD), jnp.bfloat16),
            jax.ShapeDtypeStruct((N, 2, C_PAD), jnp.float32),
        ),
        compiler_params=pltpu.CompilerParams(dimension_semantics=("parallel",)),
    )(x, w, mask)

    inv_count = 1.0 / float(N * OH * OW)
    NB = 8 if N % 8 == 0 else 1
    OH_T = next(t for t in (27, 18, 9, 6, 3, 2, 1) if OH % t == 0)
    out = pl.pallas_call(
        functools.partial(_bn_relu_kernel, eps=EPS, inv_count=inv_count,
                          OW=OW),
        grid=(N // NB, OH // OH_T),
        in_specs=[
            pl.BlockSpec((NB, OH_T, W, C_PAD), lambda nb, t: (nb, t, 0, 0)),
            pl.BlockSpec((N, 2, C_PAD), lambda nb, t: (0, 0, 0)),
            pl.BlockSpec((1, C_PAD), lambda nb, t: (0, 0)),
            pl.BlockSpec((1, C_PAD), lambda nb, t: (0, 0)),
        ],
        out_specs=pl.BlockSpec((OH_T, OW, NB, C_PAD),
                               lambda nb, t: (t, 0, nb, 0)),
        out_shape=jax.ShapeDtypeStruct((OH, OW, N, C_PAD), jnp.float32),
        compiler_params=pltpu.CompilerParams(
            dimension_semantics=("parallel", "parallel")),
    )(y, stats, g, b)
    return jnp.transpose(out[..., :COUT], (2, 3, 0, 1))


def kernel(x_nchw, w_oihw, conv_bias, gamma, beta):
    del conv_bias
    return _conv_bn_relu(x_nchw, w_oihw, gamma, beta)


# speedup vs baseline: 1.2948x; 1.2948x over previous
import functools

import jax
import jax.numpy as jnp
from jax.experimental import pallas as pl
from jax.experimental.pallas import tpu as pltpu
EPS = 1e-5
LANE = 128


def _conv_stats_kernel(x_ref, w_ref, m_ref, y_ref, stats_ref,
                       *, KH, KW, W, n_rows, pad_rows, CIN, PB):
    m = m_ref[...]
    for i in range(PB):
        xb = x_ref[i].astype(jnp.bfloat16)
        if pad_rows:
            xb = jnp.concatenate(
                [xb, jnp.zeros((pad_rows, CIN), jnp.bfloat16)], axis=0)
        taps = []
        for kh in range(KH):
            for kw in range(KW):
                off = kh * W + kw
                taps.append(xb[off:off + n_rows, :])
        patches = jnp.concatenate(taps, axis=-1)
        y = jnp.dot(patches, w_ref[...],
                    preferred_element_type=jnp.float32)
        y_ref[i] = y.astype(jnp.bfloat16).reshape(n_rows // W, W, -1)
        stats_ref[i, 0:1, :] = jnp.dot(m, y,
                                       preferred_element_type=jnp.float32)
        stats_ref[i, 1:2, :] = jnp.dot(m, y * y,
                                       preferred_element_type=jnp.float32)


def _bn_relu_kernel(y_ref, stats_ref, g_ref, b_ref, o_ref,
                    *, eps, inv_count, OW):
    tot = jnp.sum(stats_ref[...], axis=0)
    mean = tot[0:1, :] * inv_count
    var = tot[1:2, :] * inv_count - mean * mean
    inv_std = jax.lax.rsqrt(var + eps)
    scale = (g_ref[...] * inv_std).reshape(1, 1, 1, -1)
    shift = (b_ref[...] - mean * g_ref[...] * inv_std).reshape(1, 1, 1, -1)
    z = y_ref[...].astype(jnp.float32)
    z = jnp.maximum(z * scale + shift, 0.0)
    o_ref[...] = jnp.transpose(z, (1, 2, 0, 3))[:, :OW]


@jax.jit
def _conv_bn_relu(x_nchw, w_oihw, gamma, beta):
    N, CIN, H, W = x_nchw.shape
    COUT, _, KH, KW = w_oihw.shape
    OH, OW = H - KH + 1, W - KW + 1
    C_PAD = ((COUT + LANE - 1) // LANE) * LANE
    n_rows = OH * W
    HWP = -(-(H * W + KW - 1) // 8) * 8

    x = jnp.transpose(x_nchw, (0, 2, 3, 1)).reshape(N, H * W, CIN)
    w = jnp.transpose(w_oihw, (2, 3, 1, 0)).reshape(KH * KW * CIN, COUT)
    w = jnp.pad(w.astype(jnp.bfloat16), ((0, 0), (0, C_PAD - COUT)))
    g = jnp.pad(gamma.astype(jnp.float32), (0, C_PAD - COUT)).reshape(1, C_PAD)
    b = jnp.pad(beta.astype(jnp.float32), (0, C_PAD - COUT)).reshape(1, C_PAD)
    mask = (jnp.arange(n_rows) % W < OW).astype(jnp.float32).reshape(1, n_rows)

    PB = 1
    y, stats = pl.pallas_call(
        functools.partial(_conv_stats_kernel, KH=KH, KW=KW, W=W,
                          n_rows=n_rows, pad_rows=HWP - H * W, CIN=CIN, PB=PB),
        grid=(N // PB,),
        in_specs=[
            pl.BlockSpec((PB, H * W, CIN), lambda n: (n, 0, 0)),
            pl.BlockSpec((KH * KW * CIN, C_PAD), lambda n: (0, 0)),
            pl.BlockSpec((1, n_rows), lambda n: (0, 0)),
        ],
        out_specs=(
            pl.BlockSpec((PB, OH, W, C_PAD), lambda n: (n, 0, 0, 0)),
            pl.BlockSpec((PB, 2, C_PAD), lambda n: (n, 0, 0)),
        ),
        out_shape=(
            jax.ShapeDtypeStruct((N, OH, W, C_PAD), jnp.bfloat16),
            jax.ShapeDtypeStruct((N, 2, C_PAD), jnp.float32),
        ),
        compiler_params=pltpu.CompilerParams(dimension_semantics=("parallel",)),
    )(x, w, mask)

    inv_count = 1.0 / float(N * OH * OW)
    NB = 8 if N % 8 == 0 else 1
    OH_T = next(t for t in (27, 18, 9, 6, 3, 2, 1) if OH % t == 0)
    out = pl.pallas_call(
        functools.partial(_bn_relu_kernel, eps=EPS, inv_count=inv_count,
                          OW=OW),
        grid=(N // NB, OH // OH_T),
        in_specs=[
            pl.BlockSpec((NB, OH_T, W, C_PAD), lambda nb, t: (nb, t, 0, 0)),
            pl.BlockSpec((N, 2, C_PAD), lambda nb, t: (0, 0, 0)),
            pl.BlockSpec((1, C_PAD), lambda nb, t: (0, 0)),
            pl.BlockSpec((1, C_PAD), lambda nb, t: (0, 0)),
        ],
        out_specs=pl.BlockSpec((OH_T, OW, NB, C_PAD),
                               lambda nb, t: (t, 0, nb, 0)),
        out_shape=jax.ShapeDtypeStruct((OH, OW, N, C_PAD), jnp.float32),
        compiler_params=pltpu.CompilerParams(
            dimension_semantics=("parallel", "parallel")),
    )(y, stats, g, b)
    return jnp.transpose(out[..., :COUT], (2, 3, 0, 1))


def kernel(x_nchw, w_oihw, conv_bias, gamma, beta):
    del conv_bias
    return _conv_bn_relu(x_nchw, w_oihw, gamma, beta)


# speedup vs baseline: 1.4040x; 1.0843x over previous
import functools

import jax
import jax.numpy as jnp
from jax.experimental import pallas as pl
from jax.experimental.pallas import tpu as pltpu
EPS = 1e-5
LANE = 128


def _conv_stats_kernel(x_ref, w_ref, m_ref, y_ref, stats_ref,
                       *, KH, KW, W, n_rows, pad_rows, CIN, PB):
    m = m_ref[...]
    for i in range(PB):
        xb = x_ref[i]
        if pad_rows:
            xb = jnp.concatenate(
                [xb, jnp.zeros((pad_rows, CIN), jnp.float32)], axis=0)
        taps = []
        for kh in range(KH):
            for kw in range(KW):
                off = kh * W + kw
                taps.append(xb[off:off + n_rows, :])
        patches = jnp.concatenate(taps, axis=-1)
        y = jnp.dot(patches, w_ref[...],
                    precision=jax.lax.Precision.DEFAULT,
                    preferred_element_type=jnp.float32)
        y_ref[i] = y.astype(jnp.bfloat16).reshape(n_rows // W, W, -1)
        stats_ref[i, 0:1, :] = jnp.dot(m, y,
                                       preferred_element_type=jnp.float32)
        stats_ref[i, 1:2, :] = jnp.dot(m, y * y,
                                       preferred_element_type=jnp.float32)


def _bn_relu_kernel(y_ref, stats_ref, g_ref, b_ref, o_ref,
                    *, eps, inv_count, OW):
    tot = jnp.sum(stats_ref[...], axis=0)
    mean = tot[0:1, :] * inv_count
    var = tot[1:2, :] * inv_count - mean * mean
    inv_std = jax.lax.rsqrt(var + eps)
    scale = (g_ref[...] * inv_std).reshape(1, 1, 1, -1)
    shift = (b_ref[...] - mean * g_ref[...] * inv_std).reshape(1, 1, 1, -1)
    z = y_ref[...].astype(jnp.float32)
    z = jnp.maximum(z * scale + shift, 0.0)
    o_ref[...] = jnp.transpose(z, (1, 2, 0, 3))[:, :OW]


@jax.jit
def _conv_bn_relu(x_nchw, w_oihw, gamma, beta):
    N, CIN, H, W = x_nchw.shape
    COUT, _, KH, KW = w_oihw.shape
    OH, OW = H - KH + 1, W - KW + 1
    C_PAD = ((COUT + LANE - 1) // LANE) * LANE
    n_rows = OH * W
    HWP = -(-(H * W + KW - 1) // 8) * 8

    x = jnp.transpose(x_nchw, (0, 2, 3, 1)).reshape(N, H * W, CIN)
    w = jnp.transpose(w_oihw, (2, 3, 1, 0)).reshape(KH * KW * CIN, COUT)
    w = jnp.pad(w.astype(jnp.float32), ((0, 0), (0, C_PAD - COUT)))
    g = jnp.pad(gamma.astype(jnp.float32), (0, C_PAD - COUT)).reshape(1, C_PAD)
    b = jnp.pad(beta.astype(jnp.float32), (0, C_PAD - COUT)).reshape(1, C_PAD)
    mask = (jnp.arange(n_rows) % W < OW).astype(jnp.float32).reshape(1, n_rows)

    PB = 1
    y, stats = pl.pallas_call(
        functools.partial(_conv_stats_kernel, KH=KH, KW=KW, W=W,
                          n_rows=n_rows, pad_rows=HWP - H * W, CIN=CIN, PB=PB),
        grid=(N // PB,),
        in_specs=[
            pl.BlockSpec((PB, H * W, CIN), lambda n: (n, 0, 0)),
            pl.BlockSpec((KH * KW * CIN, C_PAD), lambda n: (0, 0)),
            pl.BlockSpec((1, n_rows), lambda n: (0, 0)),
        ],
        out_specs=(
            pl.BlockSpec((PB, OH, W, C_PAD), lambda n: (n, 0, 0, 0)),
            pl.BlockSpec((PB, 2, C_PAD), lambda n: (n, 0, 0)),
        ),
        out_shape=(
            jax.ShapeDtypeStruct((N, OH, W, C_PAD), jnp.bfloat16),
            jax.ShapeDtypeStruct((N, 2, C_PAD), jnp.float32),
        ),
        compiler_params=pltpu.CompilerParams(dimension_semantics=("parallel",)),
    )(x, w, mask)

    inv_count = 1.0 / float(N * OH * OW)
    NB = 8 if N % 8 == 0 else 1
    OH_T = next(t for t in (27, 18, 9, 6, 3, 2, 1) if OH % t == 0)
    out = pl.pallas_call(
        functools.partial(_bn_relu_kernel, eps=EPS, inv_count=inv_count,
                          OW=OW),
        grid=(N // NB, OH // OH_T),
        in_specs=[
            pl.BlockSpec((NB, OH_T, W, C_PAD), lambda nb, t: (nb, t, 0, 0)),
            pl.BlockSpec((N, 2, C_PAD), lambda nb, t: (0, 0, 0)),
            pl.BlockSpec((1, C_PAD), lambda nb, t: (0, 0)),
            pl.BlockSpec((1, C_PAD), lambda nb, t: (0, 0)),
        ],
        out_specs=pl.BlockSpec((OH_T, OW, NB, C_PAD),
                               lambda nb, t: (t, 0, nb, 0)),
        out_shape=jax.ShapeDtypeStruct((OH, OW, N, C_PAD), jnp.float32),
        compiler_params=pltpu.CompilerParams(
            dimension_semantics=("parallel", "parallel")),
    )(y, stats, g, b)
    return jnp.transpose(out[..., :COUT], (2, 3, 0, 1))


def kernel(x_nchw, w_oihw, conv_bias, gamma, beta):
    del conv_bias
    return _conv_bn_relu(x_nchw, w_oihw, gamma, beta)


# speedup vs baseline: 1.4175x; 1.0097x over previous
import functools

import jax
import jax.numpy as jnp
from jax.experimental import pallas as pl
from jax.experimental.pallas import tpu as pltpu
EPS = 1e-5
LANE = 128


def _conv_stats_kernel(x_ref, w_ref, m_ref, y_ref, stats_ref,
                       *, KH, KW, W, n_rows, pad_rows, CIN, PB):
    m = m_ref[...]
    for i in range(PB):
        xb = x_ref[i]
        if pad_rows:
            xb = jnp.concatenate(
                [xb, jnp.zeros((pad_rows, CIN), jnp.float32)], axis=0)
        taps = []
        for kh in range(KH):
            for kw in range(KW):
                off = kh * W + kw
                taps.append(xb[off:off + n_rows, :])
        patches = jnp.concatenate(taps, axis=-1)
        y = jnp.dot(patches, w_ref[...],
                    precision=jax.lax.Precision.DEFAULT,
                    preferred_element_type=jnp.float32)
        y_ref[i] = y.astype(jnp.bfloat16).reshape(n_rows // W, W, -1)
        stats_ref[i, 0:1, :] = jnp.dot(m, y,
                                       preferred_element_type=jnp.float32)
        stats_ref[i, 1:2, :] = jnp.dot(m, y * y,
                                       preferred_element_type=jnp.float32)


def _bn_relu_kernel(y_ref, stats_ref, g_ref, b_ref, o_ref,
                    *, eps, inv_count, OW):
    tot = jnp.sum(stats_ref[...], axis=0)
    mean = tot[0:1, :] * inv_count
    var = tot[1:2, :] * inv_count - mean * mean
    inv_std = jax.lax.rsqrt(var + eps)
    scale = (g_ref[...] * inv_std).reshape(1, 1, 1, -1)
    shift = (b_ref[...] - mean * g_ref[...] * inv_std).reshape(1, 1, 1, -1)
    z = y_ref[...].astype(jnp.float32)
    z = jnp.maximum(z * scale + shift, 0.0)
    o_ref[...] = jnp.transpose(z, (1, 2, 0, 3))[:, :OW]


@jax.jit
def _conv_bn_relu(x_nchw, w_oihw, gamma, beta):
    N, CIN, H, W = x_nchw.shape
    COUT, _, KH, KW = w_oihw.shape
    OH, OW = H - KH + 1, W - KW + 1
    C_PAD = ((COUT + LANE - 1) // LANE) * LANE
    n_rows = OH * W
    HWP = -(-(H * W + KW - 1) // 8) * 8

    x = jnp.transpose(x_nchw, (0, 2, 3, 1)).reshape(N, H * W, CIN)
    w = jnp.transpose(w_oihw, (2, 3, 1, 0)).reshape(KH * KW * CIN, COUT)
    w = jnp.pad(w.astype(jnp.float32), ((0, 0), (0, C_PAD - COUT)))
    g = jnp.pad(gamma.astype(jnp.float32), (0, C_PAD - COUT)).reshape(1, C_PAD)
    b = jnp.pad(beta.astype(jnp.float32), (0, C_PAD - COUT)).reshape(1, C_PAD)
    mask = (jnp.arange(n_rows) % W < OW).astype(jnp.float32).reshape(1, n_rows)

    PB = 1
    y, stats = pl.pallas_call(
        functools.partial(_conv_stats_kernel, KH=KH, KW=KW, W=W,
                          n_rows=n_rows, pad_rows=HWP - H * W, CIN=CIN, PB=PB),
        grid=(N // PB,),
        in_specs=[
            pl.BlockSpec((PB, H * W, CIN), lambda n: (n, 0, 0)),
            pl.BlockSpec((KH * KW * CIN, C_PAD), lambda n: (0, 0)),
            pl.BlockSpec((1, n_rows), lambda n: (0, 0)),
        ],
        out_specs=(
            pl.BlockSpec((PB, OH, W, C_PAD), lambda n: (n, 0, 0, 0)),
            pl.BlockSpec((PB, 2, C_PAD), lambda n: (n, 0, 0)),
        ),
        out_shape=(
            jax.ShapeDtypeStruct((N, OH, W, C_PAD), jnp.bfloat16),
            jax.ShapeDtypeStruct((N, 2, C_PAD), jnp.float32),
        ),
        compiler_params=pltpu.CompilerParams(dimension_semantics=("parallel",)),
    )(x, w, mask)

    inv_count = 1.0 / float(N * OH * OW)
    NB = 8 if N % 8 == 0 else 1
    OH_T = OH
    out = pl.pallas_call(
        functools.partial(_bn_relu_kernel, eps=EPS, inv_count=inv_count,
                          OW=OW),
        grid=(N // NB, OH // OH_T),
        in_specs=[
            pl.BlockSpec((NB, OH_T, W, C_PAD), lambda nb, t: (nb, t, 0, 0)),
            pl.BlockSpec((N, 2, C_PAD), lambda nb, t: (0, 0, 0)),
            pl.BlockSpec((1, C_PAD), lambda nb, t: (0, 0)),
            pl.BlockSpec((1, C_PAD), lambda nb, t: (0, 0)),
        ],
        out_specs=pl.BlockSpec((OH_T, OW, NB, C_PAD),
                               lambda nb, t: (t, 0, nb, 0)),
        out_shape=jax.ShapeDtypeStruct((OH, OW, N, C_PAD), jnp.float32),
        compiler_params=pltpu.CompilerParams(
            dimension_semantics=("parallel", "parallel")),
    )(y, stats, g, b)
    return jnp.transpose(out[..., :COUT], (2, 3, 0, 1))


def kernel(x_nchw, w_oihw, conv_bias, gamma, beta):
    del conv_bias
    return _conv_bn_relu(x_nchw, w_oihw, gamma, beta)
